# baseline (speedup 1.0000x reference)
import numpy as np
import jax
import jax.numpy as jnp
from jax.experimental import pallas as pl
from jax.experimental.pallas import tpu as pltpu
from functools import partial

_U32 = np.uint32


def _np_threefry2x32(k0, k1, x0, x1):
    x0 = np.asarray(x0, _U32).copy()
    x1 = np.asarray(x1, _U32).copy()
    k0 = _U32(k0)
    k1 = _U32(k1)
    k2 = _U32(k0 ^ k1 ^ _U32(0x1BD11BDA))
    kk = [k0, k1, k2]
    rot = [[13, 15, 26, 6], [17, 29, 16, 24]]

    def rotl(x, r):
        return ((x << _U32(r)) | (x >> _U32(32 - r))).astype(_U32)

    x0 = (x0 + k0).astype(_U32)
    x1 = (x1 + k1).astype(_U32)
    for i in range(5):
        for rr in rot[i % 2]:
            x0 = (x0 + x1).astype(_U32)
            x1 = rotl(x1, rr)
            x1 = (x1 ^ x0).astype(_U32)
        x0 = (x0 + kk[(i + 1) % 3]).astype(_U32)
        x1 = (x1 + kk[(i + 2) % 3] + _U32(i + 1)).astype(_U32)
    return x0, x1


def _derive_subkeys(seed):
    a, b = _np_threefry2x32(0, seed, np.zeros(2, _U32), np.arange(2, dtype=_U32))
    return (int(a[0]), int(b[0])), (int(a[1]), int(b[1]))


_KS, _KU = _derive_subkeys(42)

_ROT_A = (13, 15, 26, 6)
_ROT_B = (17, 29, 16, 24)


def _tf20_bits(cnt, k0i, k1i):
    u32 = jnp.uint32
    k0 = u32(k0i)
    k1 = u32(k1i)
    k2 = u32(k0i ^ k1i ^ 0x1BD11BDA)
    kk = (k0, k1, k2)
    x0 = jnp.full(cnt.shape, k0, u32)
    x1 = cnt + k1

    def rotl(x, r):
        return (x << u32(r)) | (x >> u32(32 - r))

    for i in range(5):
        rots = _ROT_A if i % 2 == 0 else _ROT_B
        for rr in rots:
            x0 = x0 + x1
            x1 = rotl(x1, rr)
            x1 = x1 ^ x0
        x0 = x0 + kk[(i + 1) % 3]
        x1 = x1 + kk[(i + 2) % 3] + u32(i + 1)
    return x0 ^ x1


def _bits_to_unit_f32(bits):
    mant = (bits >> jnp.uint32(9)) | jnp.uint32(0x3F800000)
    return jax.lax.bitcast_convert_type(mant, jnp.float32) - jnp.float32(1.0)


def _sample_kernel(pi_ref, mu_ref, samples_ref, mags_ref, logp_ref,
                   *, rows, n_op, n_aug, lanes):
    r0 = pl.program_id(0) * rows
    row = jax.lax.broadcasted_iota(jnp.int32, (rows, lanes), 0) + r0
    cat = jax.lax.broadcasted_iota(jnp.int32, (rows, lanes), 1)
    tiny = jnp.float32(np.finfo(np.float32).tiny)

    samp_cols = []
    mag_cols = []
    logp_cols = []
    for o in range(n_op):
        pi_row = pi_ref[o:o + 1, :]
        mu_row = mu_ref[0:1, :]
        mx = jnp.max(pi_row)
        lse = jnp.log(jnp.sum(jnp.exp(pi_row - mx))) + mx

        cnt = ((row * n_op + o) * n_aug + cat).astype(jnp.uint32)
        bits = _tf20_bits(cnt, _KS[0], _KS[1])
        u = jnp.maximum(_bits_to_unit_f32(bits), tiny)
        tot = -jnp.log(-jnp.log(u)) + pi_row

        m = jnp.max(tot, axis=1, keepdims=True)
        samp = jnp.min(jnp.where(tot == m, cat, jnp.int32(2**30)),
                       axis=1, keepdims=True)
        hit = cat == samp
        zero = jnp.float32(0.0)
        logit_s = jnp.sum(jnp.where(hit, pi_row, zero), axis=1, keepdims=True)
        mu_s = jnp.sum(jnp.where(hit, mu_row, zero), axis=1, keepdims=True)

        cnt2 = (row[:, :1] * n_op + o).astype(jnp.uint32)
        u2 = _bits_to_unit_f32(_tf20_bits(cnt2, _KU[0], _KU[1]))
        high = jax.nn.sigmoid(mu_s)
        logmu = -jnp.log(high)

        samp_cols.append(samp)
        mag_cols.append(u2 * high)
        logp_cols.append((logit_s - lse) + logmu)

    samples_ref[...] = jnp.concatenate(samp_cols, axis=1)
    mags_ref[...] = jnp.concatenate(mag_cols, axis=1)
    logp_ref[...] = jnp.concatenate(logp_cols, axis=1)


_B_SIZE = 16384


def _run(pi, mu, b_size):
    n_op, n_aug = pi.shape
    lanes = 1024
    rows = 128
    grid = (b_size // rows,)

    pi_pad = jnp.pad(pi, ((0, 0), (0, lanes - n_aug)),
                     constant_values=-jnp.inf)
    mu_pad = jnp.pad(mu[:, 0], (0, lanes - n_aug))[None, :]

    out_shapes = (
        jax.ShapeDtypeStruct((b_size, n_op), jnp.int32),
        jax.ShapeDtypeStruct((b_size, n_op), jnp.float32),
        jax.ShapeDtypeStruct((b_size, n_op), jnp.float32),
    )
    out_spec = pl.BlockSpec((rows, n_op), lambda i: (i, 0))
    samples, mags, logp = pl.pallas_call(
        partial(_sample_kernel, rows=rows, n_op=n_op, n_aug=n_aug,
                lanes=lanes),
        grid=grid,
        in_specs=[
            pl.BlockSpec((n_op, lanes), lambda i: (0, 0)),
            pl.BlockSpec((1, lanes), lambda i: (0, 0)),
        ],
        out_specs=(out_spec, out_spec, out_spec),
        out_shape=out_shapes,
        compiler_params=pltpu.CompilerParams(
            dimension_semantics=("parallel",)),
    )(pi_pad, mu_pad)
    return samples, mags, logp


def kernel(pi, mu, b_size):
    return _run(pi, mu, _B_SIZE)


# speedup vs baseline: 1.1051x; 1.1051x over previous
import numpy as np
import jax
import jax.numpy as jnp
from jax.experimental import pallas as pl
from jax.experimental.pallas import tpu as pltpu
from functools import partial

_U32 = np.uint32


def _np_threefry2x32(k0, k1, x0, x1):
    x0 = np.asarray(x0, _U32).copy()
    x1 = np.asarray(x1, _U32).copy()
    k0 = _U32(k0)
    k1 = _U32(k1)
    k2 = _U32(k0 ^ k1 ^ _U32(0x1BD11BDA))
    kk = [k0, k1, k2]
    rot = [[13, 15, 26, 6], [17, 29, 16, 24]]

    def rotl(x, r):
        return ((x << _U32(r)) | (x >> _U32(32 - r))).astype(_U32)

    x0 = (x0 + k0).astype(_U32)
    x1 = (x1 + k1).astype(_U32)
    for i in range(5):
        for rr in rot[i % 2]:
            x0 = (x0 + x1).astype(_U32)
            x1 = rotl(x1, rr)
            x1 = (x1 ^ x0).astype(_U32)
        x0 = (x0 + kk[(i + 1) % 3]).astype(_U32)
        x1 = (x1 + kk[(i + 2) % 3] + _U32(i + 1)).astype(_U32)
    return x0, x1


def _derive_subkeys(seed):
    a, b = _np_threefry2x32(0, seed, np.zeros(2, _U32), np.arange(2, dtype=_U32))
    return (int(a[0]), int(b[0])), (int(a[1]), int(b[1]))


_KS, _KU = _derive_subkeys(42)

_ROT_A = (13, 15, 26, 6)
_ROT_B = (17, 29, 16, 24)


def _tf20_bits(cnt, k0i, k1i):
    u32 = jnp.uint32
    k0 = u32(k0i)
    k1 = u32(k1i)
    k2 = u32(k0i ^ k1i ^ 0x1BD11BDA)
    kk = (k0, k1, k2)
    x0 = jnp.full(cnt.shape, k0, u32)
    x1 = cnt + k1

    def rotl(x, r):
        return (x << u32(r)) | (x >> u32(32 - r))

    for i in range(5):
        rots = _ROT_A if i % 2 == 0 else _ROT_B
        for rr in rots:
            x0 = x0 + x1
            x1 = rotl(x1, rr)
            x1 = x1 ^ x0
        x0 = x0 + kk[(i + 1) % 3]
        x1 = x1 + kk[(i + 2) % 3] + u32(i + 1)
    return x0 ^ x1


def _bits_to_unit_f32(bits):
    mant = (bits >> jnp.uint32(9)) | jnp.uint32(0x3F800000)
    return jax.lax.bitcast_convert_type(mant, jnp.float32) - jnp.float32(1.0)


def _sample_kernel(pi_ref, mu_ref, samples_ref, mags_ref, logp_ref,
                   *, rows, n_op, n_aug, lanes):
    r0 = pl.program_id(0) * rows
    row = jax.lax.broadcasted_iota(jnp.int32, (rows, lanes), 0) + r0
    cat = jax.lax.broadcasted_iota(jnp.int32, (rows, lanes), 1)
    tiny = jnp.float32(np.finfo(np.float32).tiny)
    mu_row = mu_ref[0:1, :]

    cnt2 = (jax.lax.broadcasted_iota(jnp.int32, (rows, n_op), 0) * n_op
            + jax.lax.broadcasted_iota(jnp.int32, (rows, n_op), 1)
            + r0 * n_op).astype(jnp.uint32)
    u2_all = _bits_to_unit_f32(_tf20_bits(cnt2, _KU[0], _KU[1]))

    cnt0 = (row * (n_op * n_aug) + cat).astype(jnp.uint32)

    samp_cols = []
    mag_cols = []
    logp_cols = []
    for o in range(n_op):
        pi_row = pi_ref[o:o + 1, :]
        mx = jnp.max(pi_row)
        lse = jnp.log(jnp.sum(jnp.exp(pi_row - mx))) + mx

        cnt = cnt0 + jnp.uint32(o * n_aug)
        bits = _tf20_bits(cnt, _KS[0], _KS[1])
        u = jnp.maximum(_bits_to_unit_f32(bits), tiny)
        tot = -jnp.log(-jnp.log(u)) + pi_row

        m = jnp.max(tot, axis=1, keepdims=True)
        samp = jnp.min(jnp.where(tot == m, cat, jnp.int32(2**30)),
                       axis=1, keepdims=True)
        hit = cat == samp
        zero = jnp.float32(0.0)
        logit_s = jnp.sum(jnp.where(hit, pi_row, zero), axis=1, keepdims=True)
        mu_s = jnp.sum(jnp.where(hit, mu_row, zero), axis=1, keepdims=True)

        high = jax.nn.sigmoid(mu_s)
        logmu = -jnp.log(high)

        samp_cols.append(samp)
        mag_cols.append(u2_all[:, o:o + 1] * high)
        logp_cols.append((logit_s - lse) + logmu)

    samples_ref[...] = jnp.concatenate(samp_cols, axis=1)
    mags_ref[...] = jnp.concatenate(mag_cols, axis=1)
    logp_ref[...] = jnp.concatenate(logp_cols, axis=1)


_B_SIZE = 16384


def _run(pi, mu, b_size):
    n_op, n_aug = pi.shape
    lanes = 1024
    rows = 256
    grid = (b_size // rows,)

    pi_pad = jnp.pad(pi, ((0, 0), (0, lanes - n_aug)),
                     constant_values=-jnp.inf)
    mu_pad = jnp.pad(mu[:, 0], (0, lanes - n_aug))[None, :]

    out_shapes = (
        jax.ShapeDtypeStruct((b_size, n_op), jnp.int32),
        jax.ShapeDtypeStruct((b_size, n_op), jnp.float32),
        jax.ShapeDtypeStruct((b_size, n_op), jnp.float32),
    )
    out_spec = pl.BlockSpec((rows, n_op), lambda i: (i, 0))
    samples, mags, logp = pl.pallas_call(
        partial(_sample_kernel, rows=rows, n_op=n_op, n_aug=n_aug,
                lanes=lanes),
        grid=grid,
        in_specs=[
            pl.BlockSpec((n_op, lanes), lambda i: (0, 0)),
            pl.BlockSpec((1, lanes), lambda i: (0, 0)),
        ],
        out_specs=(out_spec, out_spec, out_spec),
        out_shape=out_shapes,
        compiler_params=pltpu.CompilerParams(
            dimension_semantics=("parallel",)),
    )(pi_pad, mu_pad)
    return samples, mags, logp


def kernel(pi, mu, b_size):
    return _run(pi, mu, _B_SIZE)


# speedup vs baseline: 1.1455x; 1.0366x over previous
import numpy as np
import jax
import jax.numpy as jnp
from jax.experimental import pallas as pl
from jax.experimental.pallas import tpu as pltpu
from functools import partial

_U32 = np.uint32


def _np_threefry2x32(k0, k1, x0, x1):
    x0 = np.asarray(x0, _U32).copy()
    x1 = np.asarray(x1, _U32).copy()
    k0 = _U32(k0)
    k1 = _U32(k1)
    k2 = _U32(k0 ^ k1 ^ _U32(0x1BD11BDA))
    kk = [k0, k1, k2]
    rot = [[13, 15, 26, 6], [17, 29, 16, 24]]

    def rotl(x, r):
        return ((x << _U32(r)) | (x >> _U32(32 - r))).astype(_U32)

    x0 = (x0 + k0).astype(_U32)
    x1 = (x1 + k1).astype(_U32)
    for i in range(5):
        for rr in rot[i % 2]:
            x0 = (x0 + x1).astype(_U32)
            x1 = rotl(x1, rr)
            x1 = (x1 ^ x0).astype(_U32)
        x0 = (x0 + kk[(i + 1) % 3]).astype(_U32)
        x1 = (x1 + kk[(i + 2) % 3] + _U32(i + 1)).astype(_U32)
    return x0, x1


def _derive_subkeys(seed):
    a, b = _np_threefry2x32(0, seed, np.zeros(2, _U32), np.arange(2, dtype=_U32))
    return (int(a[0]), int(b[0])), (int(a[1]), int(b[1]))


_KS, _KU = _derive_subkeys(42)

_ROT_A = (13, 15, 26, 6)
_ROT_B = (17, 29, 16, 24)


def _tf20_bits(cnt, k0i, k1i):
    u32 = jnp.uint32
    k2i = (k0i ^ k1i ^ 0x1BD11BDA) & 0xFFFFFFFF
    kk = (k0i, k1i, k2i)
    x0 = jnp.full(cnt.shape, u32(k0i), u32)
    x1 = cnt + u32(k1i)

    def rotl(x, r):
        return (x << u32(r)) | (x >> u32(32 - r))

    for i in range(5):
        rots = _ROT_A if i % 2 == 0 else _ROT_B
        for rr in rots:
            x0 = x0 + x1
            x1 = rotl(x1, rr)
            x1 = x1 ^ x0
        x0 = x0 + u32(kk[(i + 1) % 3])
        x1 = x1 + u32((kk[(i + 2) % 3] + i + 1) & 0xFFFFFFFF)
    return x0 ^ x1


def _bits_to_unit_f32(bits):
    mant = (bits >> jnp.uint32(9)) | jnp.uint32(0x3F800000)
    return jax.lax.bitcast_convert_type(mant, jnp.float32) - jnp.float32(1.0)


def _sample_kernel(pi_ref, mu_ref, samples_ref, mags_ref, logp_ref,
                   *, rows, n_op, n_aug, lanes):
    r0 = pl.program_id(0) * rows
    row = jax.lax.broadcasted_iota(jnp.int32, (rows, lanes), 0) + r0
    cat = jax.lax.broadcasted_iota(jnp.int32, (rows, lanes), 1)
    tiny = jnp.float32(np.finfo(np.float32).tiny)
    mu_row = mu_ref[0:1, :]

    cnt2 = (jax.lax.broadcasted_iota(jnp.int32, (rows, n_op), 0) * n_op
            + jax.lax.broadcasted_iota(jnp.int32, (rows, n_op), 1)
            + r0 * n_op).astype(jnp.uint32)
    u2_all = _bits_to_unit_f32(_tf20_bits(cnt2, _KU[0], _KU[1]))

    cnt0 = (row * (n_op * n_aug) + cat).astype(jnp.uint32)

    samp_cols = []
    mag_cols = []
    logp_cols = []
    for o in range(n_op):
        pi_row = pi_ref[o:o + 1, :]
        mx = jnp.max(pi_row)
        lse = jnp.log(jnp.sum(jnp.exp(pi_row - mx))) + mx

        cnt = cnt0 + jnp.uint32(o * n_aug)
        bits = _tf20_bits(cnt, _KS[0], _KS[1])
        u = jnp.maximum(_bits_to_unit_f32(bits), tiny)
        tot = -jnp.log(-jnp.log(u)) + pi_row

        m = jnp.max(tot, axis=1, keepdims=True)
        samp = jnp.min(jnp.where(tot == m, cat, jnp.int32(2**30)),
                       axis=1, keepdims=True)
        hit = cat == samp
        zero = jnp.float32(0.0)
        logit_s = jnp.sum(jnp.where(hit, pi_row, zero), axis=1, keepdims=True)
        mu_s = jnp.sum(jnp.where(hit, mu_row, zero), axis=1, keepdims=True)

        high = jax.nn.sigmoid(mu_s)
        logmu = -jnp.log(high)

        samp_cols.append(samp)
        mag_cols.append(u2_all[:, o:o + 1] * high)
        logp_cols.append((logit_s - lse) + logmu)

    samples_ref[...] = jnp.concatenate(samp_cols, axis=1)
    mags_ref[...] = jnp.concatenate(mag_cols, axis=1)
    logp_ref[...] = jnp.concatenate(logp_cols, axis=1)


_B_SIZE = 16384


def _run(pi, mu, b_size):
    n_op, n_aug = pi.shape
    lanes = 1024
    rows = 256
    grid = (b_size // rows,)

    pi_pad = jnp.pad(pi, ((0, 0), (0, lanes - n_aug)),
                     constant_values=-jnp.inf)
    mu_pad = jnp.pad(mu[:, 0], (0, lanes - n_aug))[None, :]

    out_shapes = (
        jax.ShapeDtypeStruct((b_size, n_op), jnp.int32),
        jax.ShapeDtypeStruct((b_size, n_op), jnp.float32),
        jax.ShapeDtypeStruct((b_size, n_op), jnp.float32),
    )
    out_spec = pl.BlockSpec((rows, n_op), lambda i: (i, 0))
    samples, mags, logp = pl.pallas_call(
        partial(_sample_kernel, rows=rows, n_op=n_op, n_aug=n_aug,
                lanes=lanes),
        grid=grid,
        in_specs=[
            pl.BlockSpec((n_op, lanes), lambda i: (0, 0)),
            pl.BlockSpec((1, lanes), lambda i: (0, 0)),
        ],
        out_specs=(out_spec, out_spec, out_spec),
        out_shape=out_shapes,
        compiler_params=pltpu.CompilerParams(
            dimension_semantics=("parallel",)),
    )(pi_pad, mu_pad)
    return samples, mags, logp


def kernel(pi, mu, b_size):
    return _run(pi, mu, _B_SIZE)
